# Initial kernel scaffold; baseline (speedup 1.0000x reference)
import functools

import jax
import jax.numpy as jnp
from jax import lax
from jax.experimental import pallas as pl
from jax.experimental.pallas import tpu as pltpu
from jax.experimental.pallas import tpu_sc as plsc

VOCAB = 100000
EMBED_DIM = 64
NUM_TOKENS = 819200

_NC = 2
_NS = 16
_NW = _NC * _NS
_B_PER_W = NUM_TOKENS // _NW
_CHUNK = 1024
_N_CHUNKS = _B_PER_W // _CHUNK

_mesh = plsc.VectorSubcoreMesh(core_axis_name="c", subcore_axis_name="s")


@functools.partial(
    pl.kernel,
    mesh=_mesh,
    out_type=jax.ShapeDtypeStruct((NUM_TOKENS, EMBED_DIM), jnp.float32),
    scratch_types=[
        pltpu.VMEM((_CHUNK,), jnp.int32),
        pltpu.VMEM((_CHUNK, EMBED_DIM), jnp.float32),
        pltpu.SemaphoreType.DMA,
    ],
)
def _gather_kernel(idx_hbm, table_hbm, out_hbm, idx_v, rows_v, sem):
    wid = lax.axis_index("s") * _NC + lax.axis_index("c")
    base = wid * _B_PER_W

    def body(i, carry):
        off = base + i * _CHUNK
        pltpu.sync_copy(idx_hbm.at[pl.ds(off, _CHUNK)], idx_v)
        pltpu.async_copy(table_hbm.at[idx_v], rows_v, sem).wait()
        pltpu.sync_copy(rows_v, out_hbm.at[pl.ds(off, _CHUNK)])
        return carry

    lax.fori_loop(0, _N_CHUNKS, body, 0)


def kernel(word_sequence, table):
    idx = word_sequence.astype(jnp.int32)
    return _gather_kernel(idx, table)


# speedup vs baseline: 4.6414x; 4.6414x over previous
import functools

import jax
import jax.numpy as jnp
from jax import lax
from jax.experimental import pallas as pl
from jax.experimental.pallas import tpu as pltpu
from jax.experimental.pallas import tpu_sc as plsc

VOCAB = 100000
EMBED_DIM = 64
NUM_TOKENS = 819200

_NC = 2
_NS = 16
_NW = _NC * _NS
_B_PER_W = NUM_TOKENS // _NW
_CHUNK = 800
_N_CHUNKS = _B_PER_W // _CHUNK

_mesh = plsc.VectorSubcoreMesh(core_axis_name="c", subcore_axis_name="s")


@functools.partial(
    pl.kernel,
    mesh=_mesh,
    out_type=jax.ShapeDtypeStruct((NUM_TOKENS, EMBED_DIM), jnp.float32),
    scratch_types=[
        pltpu.VMEM((_CHUNK,), jnp.int32),
        pltpu.VMEM((_CHUNK,), jnp.int32),
        pltpu.VMEM((_CHUNK, EMBED_DIM), jnp.float32),
        pltpu.VMEM((_CHUNK, EMBED_DIM), jnp.float32),
        pltpu.SemaphoreType.DMA,
        pltpu.SemaphoreType.DMA,
        pltpu.SemaphoreType.DMA,
        pltpu.SemaphoreType.DMA,
    ],
    compiler_params=pltpu.CompilerParams(use_tc_tiling_on_sc=False),
)
def _gather_kernel(idx_hbm, table_hbm, out_hbm,
                   idx_v0, idx_v1, rows_v0, rows_v1,
                   sem_g0, sem_g1, sem_w0, sem_w1):
    wid = lax.axis_index("s") * _NC + lax.axis_index("c")
    base = wid * _B_PER_W
    bufs = ((idx_v0, rows_v0, sem_g0, sem_w0),
            (idx_v1, rows_v1, sem_g1, sem_w1))

    for b in range(2):
        idx_v, rows_v, sem_g, _ = bufs[b]
        off = base + b * _CHUNK
        pltpu.sync_copy(idx_hbm.at[pl.ds(off, _CHUNK)], idx_v)
        pltpu.async_copy(table_hbm.at[idx_v], rows_v, sem_g)

    def body(g, carry):
        for b in range(2):
            idx_v, rows_v, sem_g, sem_w = bufs[b]
            i = 2 * g + b
            off = base + i * _CHUNK
            pltpu.make_async_copy(table_hbm.at[idx_v], rows_v, sem_g).wait()
            pltpu.async_copy(rows_v, out_hbm.at[pl.ds(off, _CHUNK)], sem_w)
            off2 = off + 2 * _CHUNK
            pltpu.sync_copy(idx_hbm.at[pl.ds(off2, _CHUNK)], idx_v)
            pltpu.make_async_copy(rows_v, out_hbm.at[pl.ds(off, _CHUNK)],
                                  sem_w).wait()
            pltpu.async_copy(table_hbm.at[idx_v], rows_v, sem_g)
        return carry

    lax.fori_loop(0, _N_CHUNKS // 2 - 1, body, 0)

    for b in range(2):
        idx_v, rows_v, sem_g, sem_w = bufs[b]
        i = _N_CHUNKS - 2 + b
        off = base + i * _CHUNK
        pltpu.make_async_copy(table_hbm.at[idx_v], rows_v, sem_g).wait()
        pltpu.async_copy(rows_v, out_hbm.at[pl.ds(off, _CHUNK)], sem_w)
    for b in range(2):
        idx_v, rows_v, _, sem_w = bufs[b]
        i = _N_CHUNKS - 2 + b
        off = base + i * _CHUNK
        pltpu.make_async_copy(rows_v, out_hbm.at[pl.ds(off, _CHUNK)],
                              sem_w).wait()


def kernel(word_sequence, table):
    idx = word_sequence.astype(jnp.int32)
    return _gather_kernel(idx, table)
